# baseline (speedup 1.0000x reference)
import functools

import jax
import jax.numpy as jnp
from jax.experimental import pallas as pl
from jax.experimental.pallas import tpu as pltpu

TOPK = 10
CW = 64
NEG = float(-3e38)
BIGF = float(3e38)
BIGCOL = 2**30


def _topk_body(q_ref, k_ref, out_s_ref, out_i_ref,
               t_ref, ti_ref, cm_ref, ca_ref, mv_ref, mc_ref,
               *, nk, kb, n_keys):
    ki = pl.program_id(0)
    nch = kb // CW

    @pl.when(ki == 0)
    def _init():
        t_ref[...] = jnp.full_like(t_ref, NEG)
        ti_ref[...] = jnp.zeros_like(ti_ref)

    s = jax.lax.dot_general(
        q_ref[...], k_ref[...],
        (((1,), (1,)), ((), ())),
        preferred_element_type=jnp.float32,
    )
    rows = s.shape[0]
    col = jax.lax.broadcasted_iota(jnp.int32, (rows, kb), 1)
    base = ki * kb
    s = jnp.where(base + col < n_keys, s, NEG)

    lane = jax.lax.broadcasted_iota(jnp.int32, (rows, nch), 1)
    iota_t = jax.lax.broadcasted_iota(jnp.int32, (1, TOPK), 1)

    def chunk_candidate(c, with_exclusion):
        sc = s[:, c * CW:(c + 1) * CW]
        cc = col[:, c * CW:(c + 1) * CW]
        if with_exclusion:
            mvc = mv_ref[:, c:c + 1]
            mcc = mc_ref[:, c:c + 1]
            live = (sc < mvc) | ((sc == mvc) & (cc > mcc))
            sc = jnp.where(live, sc, NEG)
        m = jnp.max(sc, axis=1, keepdims=True)
        a = jnp.min(jnp.where(sc == m, cc, BIGCOL), axis=1, keepdims=True)
        return m, a

    mv_ref[...] = jnp.full_like(mv_ref, BIGF)
    mc_ref[...] = jnp.full_like(mc_ref, -1)
    cands = [chunk_candidate(c, with_exclusion=False) for c in range(nch)]
    cm_ref[...] = jnp.concatenate([m for m, _ in cands], axis=1)
    ca_ref[...] = jnp.concatenate([a for _, a in cands], axis=1)

    def merge_one(_):
        cm = cm_ref[...]
        t = t_ref[...]
        ti = ti_ref[...]
        t9 = t[:, TOPK - 1:TOPK]
        m = jnp.max(cm, axis=1, keepdims=True)
        wc = jnp.min(jnp.where(cm == m, lane, BIGCOL), axis=1, keepdims=True)
        ca = jnp.min(jnp.where(lane == wc, ca_ref[...], BIGCOL), axis=1,
                     keepdims=True)
        upd = m > t9
        gm = base + ca
        p = jnp.sum(((t > m) | ((t == m) & (ti < gm))).astype(jnp.int32),
                    axis=1, keepdims=True)
        t_sh = jnp.concatenate([t[:, :1], t[:, :-1]], axis=1)
        ti_sh = jnp.concatenate([ti[:, :1], ti[:, :-1]], axis=1)
        t_ref[...] = jnp.where(iota_t < p, t, jnp.where(iota_t == p, m, t_sh))
        ti_ref[...] = jnp.where(iota_t < p, ti,
                                jnp.where(iota_t == p, gm, ti_sh))
        sel = (lane == wc) & upd
        mv_ref[...] = jnp.where(sel, m, mv_ref[...])
        mc_ref[...] = jnp.where(sel, ca, mc_ref[...])
        cm_new = jnp.where(sel, NEG, cm)
        cm_ref[...] = cm_new
        m2 = jnp.max(cm_new, axis=1, keepdims=True)
        return jnp.any(m2 > t_ref[:, TOPK - 1:TOPK])

    def have_work():
        m = jnp.max(cm_ref[...], axis=1, keepdims=True)
        return jnp.any(m > t_ref[:, TOPK - 1:TOPK])

    def outer_body(_):
        jax.lax.while_loop(lambda g: g, merge_one, True)
        consumed = jnp.min(cm_ref[...], axis=0, keepdims=True)
        for c in range(nch):
            flag = jnp.min(consumed[:, c:c + 1])

            @pl.when(flag == NEG)
            def _refresh(c=c):
                m, a = chunk_candidate(c, with_exclusion=True)
                cm_ref[:, c:c + 1] = m
                ca_ref[:, c:c + 1] = a
        return have_work()

    jax.lax.while_loop(lambda g: g, outer_body, have_work())

    @pl.when(ki == nk - 1)
    def _emit():
        out_s_ref[...] = t_ref[...]
        out_i_ref[...] = ti_ref[...]


def kernel(queries, keys):
    n_q, dim = queries.shape
    n_keys, _ = keys.shape
    kb = min(2048, n_keys)
    nk = pl.cdiv(n_keys, kb)

    body = functools.partial(_topk_body, nk=nk, kb=kb, n_keys=n_keys)
    out_s, out_i = pl.pallas_call(
        body,
        grid=(nk,),
        in_specs=[
            pl.BlockSpec((n_q, dim), lambda ki: (0, 0)),
            pl.BlockSpec((kb, dim), lambda ki: (ki, 0)),
        ],
        out_specs=[
            pl.BlockSpec((n_q, TOPK), lambda ki: (0, 0)),
            pl.BlockSpec((n_q, TOPK), lambda ki: (0, 0)),
        ],
        out_shape=[
            jax.ShapeDtypeStruct((n_q, TOPK), jnp.float32),
            jax.ShapeDtypeStruct((n_q, TOPK), jnp.int32),
        ],
        scratch_shapes=[
            pltpu.VMEM((n_q, TOPK), jnp.float32),
            pltpu.VMEM((n_q, TOPK), jnp.int32),
            pltpu.VMEM((n_q, kb // CW), jnp.float32),
            pltpu.VMEM((n_q, kb // CW), jnp.int32),
            pltpu.VMEM((n_q, kb // CW), jnp.float32),
            pltpu.VMEM((n_q, kb // CW), jnp.int32),
        ],
        compiler_params=pltpu.CompilerParams(
            dimension_semantics=("arbitrary",),
        ),
    )(queries, keys)
    return out_s, out_i


# speedup vs baseline: 3.5106x; 3.5106x over previous
import functools

import jax
import jax.numpy as jnp
from jax.experimental import pallas as pl
from jax.experimental.pallas import tpu as pltpu

TOPK = 10
NEG = float(-3e38)
BIGCOL = 2**30


def _topk_body(q_ref, k_ref, out_s_ref, out_i_ref, t_ref, ti_ref,
               *, nk, kb, n_keys):
    ki = pl.program_id(0)

    @pl.when(ki == 0)
    def _init():
        t_ref[...] = jnp.full_like(t_ref, NEG)
        ti_ref[...] = jnp.zeros_like(ti_ref)

    s = jax.lax.dot_general(
        q_ref[...], k_ref[...],
        (((1,), (1,)), ((), ())),
        preferred_element_type=jnp.float32,
    )
    rows = s.shape[0]
    gr = min(128, rows)
    ngr = rows // gr
    base = ki * kb
    col_full = jax.lax.broadcasted_iota(jnp.int32, (rows, kb), 1)
    s = jnp.where(base + col_full < n_keys, s, NEG)
    col = jax.lax.broadcasted_iota(jnp.int32, (gr, kb), 1)

    iota_t = jax.lax.broadcasted_iota(jnp.int32, (1, TOPK), 1)

    for g in range(ngr):
        rs = g * gr
        sg = s[rs:rs + gr, :]

        def first_max(sg=sg):
            m = jnp.max(sg, axis=1, keepdims=True)
            am = jnp.min(jnp.where(sg == m, col, BIGCOL), axis=1,
                         keepdims=True)
            return m, am

        def cond(carry):
            m, _, t, _ = carry
            return jnp.any(m > t[:, TOPK - 1:TOPK])

        def body(carry, sg=sg):
            m, am, t, ti = carry
            gm = base + am
            p = jnp.sum(((t > m) | ((t == m) & (ti < gm))).astype(jnp.int32),
                        axis=1, keepdims=True)
            t_sh = jnp.concatenate([t[:, :1], t[:, :-1]], axis=1)
            ti_sh = jnp.concatenate([ti[:, :1], ti[:, :-1]], axis=1)
            t = jnp.where(iota_t < p, t, jnp.where(iota_t == p, m, t_sh))
            ti = jnp.where(iota_t < p, ti, jnp.where(iota_t == p, gm, ti_sh))
            live = (sg < m) | ((sg == m) & (col > am))
            sm = jnp.where(live, sg, NEG)
            m2 = jnp.max(sm, axis=1, keepdims=True)
            am2 = jnp.min(jnp.where(sm == m2, col, BIGCOL), axis=1,
                          keepdims=True)
            return m2, am2, t, ti

        m0, am0 = first_max()
        _, _, t, ti = jax.lax.while_loop(
            cond, body, (m0, am0, t_ref[rs:rs + gr, :], ti_ref[rs:rs + gr, :]))
        t_ref[rs:rs + gr, :] = t
        ti_ref[rs:rs + gr, :] = ti

    @pl.when(ki == nk - 1)
    def _emit():
        out_s_ref[...] = t_ref[...]
        out_i_ref[...] = ti_ref[...]


def kernel(queries, keys):
    n_q, dim = queries.shape
    n_keys, _ = keys.shape
    kb = min(2048, n_keys)
    nk = pl.cdiv(n_keys, kb)

    body = functools.partial(_topk_body, nk=nk, kb=kb, n_keys=n_keys)
    out_s, out_i = pl.pallas_call(
        body,
        grid=(nk,),
        in_specs=[
            pl.BlockSpec((n_q, dim), lambda ki: (0, 0)),
            pl.BlockSpec((kb, dim), lambda ki: (ki, 0)),
        ],
        out_specs=[
            pl.BlockSpec((n_q, TOPK), lambda ki: (0, 0)),
            pl.BlockSpec((n_q, TOPK), lambda ki: (0, 0)),
        ],
        out_shape=[
            jax.ShapeDtypeStruct((n_q, TOPK), jnp.float32),
            jax.ShapeDtypeStruct((n_q, TOPK), jnp.int32),
        ],
        scratch_shapes=[
            pltpu.VMEM((n_q, TOPK), jnp.float32),
            pltpu.VMEM((n_q, TOPK), jnp.int32),
        ],
        compiler_params=pltpu.CompilerParams(
            dimension_semantics=("arbitrary",),
        ),
    )(queries, keys)
    return out_s, out_i


# speedup vs baseline: 4.2627x; 1.2142x over previous
import functools

import jax
import jax.numpy as jnp
from jax.experimental import pallas as pl
from jax.experimental.pallas import tpu as pltpu

TOPK = 10
STATIC_E = 2
NEG = float(-3e38)
BIGCOL = 2**30


def _topk_body(q_ref, k_ref, out_s_ref, out_i_ref, t_ref, ti_ref, sbuf_ref,
               *, nk, kb, n_keys):
    ki = pl.program_id(0)
    rows = q_ref.shape[0]
    col = jax.lax.broadcasted_iota(jnp.int32, (rows, kb), 1)
    iota_t = jax.lax.broadcasted_iota(jnp.int32, (1, TOPK), 1)

    @pl.when(ki == 0)
    def _init():
        t_ref[...] = jnp.full_like(t_ref, NEG)
        ti_ref[...] = jnp.zeros_like(ti_ref)

    s = sbuf_ref[...]

    @pl.when(ki < nk)
    def _compute():
        s_new = jax.lax.dot_general(
            q_ref[...], k_ref[...],
            (((1,), (1,)), ((), ())),
            preferred_element_type=jnp.float32,
        )
        sbuf_ref[...] = jnp.where(ki * kb + col < n_keys, s_new, NEG)

    @pl.when(ki > 0)
    def _merge():
        base = (ki - 1) * kb

        def step(carry):
            m, am, t, ti = carry
            gm = base + am
            p = jnp.sum(((t > m) | ((t == m) & (ti < gm))).astype(jnp.int32),
                        axis=1, keepdims=True)
            t_sh = jnp.concatenate([t[:, :1], t[:, :-1]], axis=1)
            ti_sh = jnp.concatenate([ti[:, :1], ti[:, :-1]], axis=1)
            t = jnp.where(iota_t < p, t, jnp.where(iota_t == p, m, t_sh))
            ti = jnp.where(iota_t < p, ti, jnp.where(iota_t == p, gm, ti_sh))
            live = (s < m) | ((s == m) & (col > am))
            sm = jnp.where(live, s, NEG)
            m2 = jnp.max(sm, axis=1, keepdims=True)
            am2 = jnp.min(jnp.where(sm == m2, col, BIGCOL), axis=1,
                          keepdims=True)
            return m2, am2, t, ti

        def cond(carry):
            m, _, t, _ = carry
            return jnp.any(m > t[:, TOPK - 1:TOPK])

        m0 = jnp.max(s, axis=1, keepdims=True)
        am0 = jnp.min(jnp.where(s == m0, col, BIGCOL), axis=1, keepdims=True)
        carry = (m0, am0, t_ref[...], ti_ref[...])
        for _ in range(STATIC_E):
            carry = step(carry)
        m, am, t, ti = jax.lax.while_loop(cond, step, carry)
        t_ref[...] = t
        ti_ref[...] = ti

    @pl.when(ki == nk)
    def _emit():
        out_s_ref[...] = t_ref[...]
        out_i_ref[...] = ti_ref[...]


def kernel(queries, keys):
    n_q, dim = queries.shape
    n_keys, _ = keys.shape
    kb = min(2048, n_keys)
    nk = pl.cdiv(n_keys, kb)

    body = functools.partial(_topk_body, nk=nk, kb=kb, n_keys=n_keys)
    out_s, out_i = pl.pallas_call(
        body,
        grid=(nk + 1,),
        in_specs=[
            pl.BlockSpec((n_q, dim), lambda ki: (0, 0)),
            pl.BlockSpec((kb, dim), lambda ki: (jnp.minimum(ki, nk - 1), 0)),
        ],
        out_specs=[
            pl.BlockSpec((n_q, TOPK), lambda ki: (0, 0)),
            pl.BlockSpec((n_q, TOPK), lambda ki: (0, 0)),
        ],
        out_shape=[
            jax.ShapeDtypeStruct((n_q, TOPK), jnp.float32),
            jax.ShapeDtypeStruct((n_q, TOPK), jnp.int32),
        ],
        scratch_shapes=[
            pltpu.VMEM((n_q, TOPK), jnp.float32),
            pltpu.VMEM((n_q, TOPK), jnp.int32),
            pltpu.VMEM((n_q, kb), jnp.float32),
        ],
        compiler_params=pltpu.CompilerParams(
            dimension_semantics=("arbitrary",),
        ),
    )(queries, keys)
    return out_s, out_i


# speedup vs baseline: 4.5804x; 1.0745x over previous
import functools

import jax
import jax.numpy as jnp
from jax.experimental import pallas as pl
from jax.experimental.pallas import tpu as pltpu

TOPK = 10
STATIC_E = 2
NEG = float(-3e38)
BIGCOL = float(2**30)


def _topk_body(q_ref, k_ref, out_s_ref, out_i_ref, t_ref, ti_ref, sbuf_ref,
               *, nk, kb, n_keys):
    ki = pl.program_id(0)
    rows = q_ref.shape[0]
    col = jax.lax.broadcasted_iota(jnp.int32, (rows, kb), 1).astype(jnp.float32)
    iota_t = jax.lax.broadcasted_iota(jnp.int32, (1, TOPK), 1)

    @pl.when(ki == 0)
    def _init():
        t_ref[...] = jnp.full_like(t_ref, NEG)
        ti_ref[...] = jnp.zeros_like(ti_ref)

    s = sbuf_ref[...]

    @pl.when(ki < nk)
    def _compute():
        s_new = jax.lax.dot_general(
            q_ref[...], k_ref[...],
            (((1,), (1,)), ((), ())),
            preferred_element_type=jnp.float32,
        )
        sbuf_ref[...] = jnp.where(ki * kb + col < n_keys, s_new, NEG)

    @pl.when(ki > 0)
    def _merge():
        base = (ki - 1) * kb

        def step(carry):
            m, am, t, ti = carry
            gm = base + am.astype(jnp.int32)
            p = jnp.sum(((t > m) | ((t == m) & (ti < gm))).astype(jnp.int32),
                        axis=1, keepdims=True)
            t_sh = jnp.concatenate([t[:, :1], t[:, :-1]], axis=1)
            ti_sh = jnp.concatenate([ti[:, :1], ti[:, :-1]], axis=1)
            t = jnp.where(iota_t < p, t, jnp.where(iota_t == p, m, t_sh))
            ti = jnp.where(iota_t < p, ti, jnp.where(iota_t == p, gm, ti_sh))
            live = (s < m) | ((s == m) & (col > am))
            sm = jnp.where(live, s, NEG)
            m2 = jnp.max(sm, axis=1, keepdims=True)
            am2 = jnp.min(jnp.where(sm == m2, col, BIGCOL), axis=1,
                          keepdims=True)
            return m2, am2, t, ti

        def cond(carry):
            m, _, t, _ = carry
            return jnp.any(m > t[:, TOPK - 1:TOPK])

        m0 = jnp.max(s, axis=1, keepdims=True)
        am0 = jnp.min(jnp.where(s == m0, col, BIGCOL), axis=1, keepdims=True)
        carry = (m0, am0, t_ref[...], ti_ref[...])
        for _ in range(STATIC_E):
            carry = step(carry)
        m, am, t, ti = jax.lax.while_loop(cond, step, carry)
        t_ref[...] = t
        ti_ref[...] = ti

    @pl.when(ki == nk)
    def _emit():
        out_s_ref[...] = t_ref[...]
        out_i_ref[...] = ti_ref[...]


def kernel(queries, keys):
    n_q, dim = queries.shape
    n_keys, _ = keys.shape
    kb = min(2048, n_keys)
    nk = pl.cdiv(n_keys, kb)

    body = functools.partial(_topk_body, nk=nk, kb=kb, n_keys=n_keys)
    out_s, out_i = pl.pallas_call(
        body,
        grid=(nk + 1,),
        in_specs=[
            pl.BlockSpec((n_q, dim), lambda ki: (0, 0)),
            pl.BlockSpec((kb, dim), lambda ki: (jnp.minimum(ki, nk - 1), 0)),
        ],
        out_specs=[
            pl.BlockSpec((n_q, TOPK), lambda ki: (0, 0)),
            pl.BlockSpec((n_q, TOPK), lambda ki: (0, 0)),
        ],
        out_shape=[
            jax.ShapeDtypeStruct((n_q, TOPK), jnp.float32),
            jax.ShapeDtypeStruct((n_q, TOPK), jnp.int32),
        ],
        scratch_shapes=[
            pltpu.VMEM((n_q, TOPK), jnp.float32),
            pltpu.VMEM((n_q, TOPK), jnp.int32),
            pltpu.VMEM((n_q, kb), jnp.float32),
        ],
        compiler_params=pltpu.CompilerParams(
            dimension_semantics=("arbitrary",),
        ),
    )(queries, keys)
    return out_s, out_i


# speedup vs baseline: 4.6479x; 1.0148x over previous
import functools

import jax
import jax.numpy as jnp
from jax.experimental import pallas as pl
from jax.experimental.pallas import tpu as pltpu

TOPK = 10
STATIC_E = 2
NEG = float(-3e38)
BIGCOL = float(2**30)


def _topk_body(q_ref, k_ref, out_s_ref, out_i_ref, t_ref, ti_ref, sbuf_ref,
               *, nk, kb, n_keys):
    ki = pl.program_id(0)
    rows = q_ref.shape[0]
    col = jax.lax.broadcasted_iota(jnp.int32, (rows, kb), 1).astype(jnp.float32)
    iota_t = jax.lax.broadcasted_iota(jnp.int32, (1, TOPK), 1)

    @pl.when(ki == 0)
    def _init():
        t_ref[...] = jnp.full_like(t_ref, NEG)
        ti_ref[...] = jnp.zeros_like(ti_ref)

    s = sbuf_ref[...]

    @pl.when(ki < nk)
    def _compute():
        s_new = jax.lax.dot_general(
            q_ref[...], k_ref[...],
            (((1,), (1,)), ((), ())),
            preferred_element_type=jnp.float32,
        )
        sbuf_ref[...] = jnp.where(ki * kb + col < n_keys, s_new, NEG)

    @pl.when(ki > 0)
    def _merge():
        base = (ki - 1) * kb

        def step(carry):
            m, am, t, ti = carry
            gm = base + am.astype(jnp.int32)
            p = jnp.sum(((t > m) | ((t == m) & (ti < gm))).astype(jnp.int32),
                        axis=1, keepdims=True)
            t_sh = jnp.concatenate([t[:, :1], t[:, :-1]], axis=1)
            ti_sh = jnp.concatenate([ti[:, :1], ti[:, :-1]], axis=1)
            t = jnp.where(iota_t < p, t, jnp.where(iota_t == p, m, t_sh))
            ti = jnp.where(iota_t < p, ti, jnp.where(iota_t == p, gm, ti_sh))
            live = (s < m) | ((s == m) & (col > am))
            sm = jnp.where(live, s, NEG)
            m2 = jnp.max(sm, axis=1, keepdims=True)
            am2 = jnp.min(jnp.where(sm == m2, col, BIGCOL), axis=1,
                          keepdims=True)
            return m2, am2, t, ti

        def cond(carry):
            m, _, t, _ = carry
            return jnp.any(m > t[:, TOPK - 1:TOPK])

        m0 = jnp.max(s, axis=1, keepdims=True)
        am0 = jnp.min(jnp.where(s == m0, col, BIGCOL), axis=1, keepdims=True)
        carry = (m0, am0, t_ref[...], ti_ref[...])
        for _ in range(STATIC_E):
            carry = step(carry)
        m, am, t, ti = jax.lax.while_loop(cond, step, carry)
        t_ref[...] = t
        ti_ref[...] = ti

    @pl.when(ki == nk)
    def _emit():
        out_s_ref[...] = t_ref[...]
        out_i_ref[...] = ti_ref[...]


def kernel(queries, keys):
    n_q, dim = queries.shape
    n_keys, _ = keys.shape
    kb = min(1024, n_keys)
    nk = pl.cdiv(n_keys, kb)

    body = functools.partial(_topk_body, nk=nk, kb=kb, n_keys=n_keys)
    out_s, out_i = pl.pallas_call(
        body,
        grid=(nk + 1,),
        in_specs=[
            pl.BlockSpec((n_q, dim), lambda ki: (0, 0)),
            pl.BlockSpec((kb, dim), lambda ki: (jnp.minimum(ki, nk - 1), 0)),
        ],
        out_specs=[
            pl.BlockSpec((n_q, TOPK), lambda ki: (0, 0)),
            pl.BlockSpec((n_q, TOPK), lambda ki: (0, 0)),
        ],
        out_shape=[
            jax.ShapeDtypeStruct((n_q, TOPK), jnp.float32),
            jax.ShapeDtypeStruct((n_q, TOPK), jnp.int32),
        ],
        scratch_shapes=[
            pltpu.VMEM((n_q, TOPK), jnp.float32),
            pltpu.VMEM((n_q, TOPK), jnp.int32),
            pltpu.VMEM((n_q, kb), jnp.float32),
        ],
        compiler_params=pltpu.CompilerParams(
            dimension_semantics=("arbitrary",),
        ),
    )(queries, keys)
    return out_s, out_i
